# baseline (speedup 1.0000x reference)
import functools

import jax
import jax.numpy as jnp
from jax.experimental import pallas as pl
from jax.experimental.pallas import tpu as pltpu

TN = 1024
EPR = 4


def _expert_of(ei):
    round_idx = ei // (EPR + 1)
    within = ei % (EPR + 1)
    is_m2 = within == EPR
    exp = round_idx * EPR + jnp.minimum(within, EPR - 1)
    return exp, is_m2, round_idx, within


def _body(x_ref, dp_ref, cb_ref, dm_ref, w1_ref, b1_ref, w2_ref, b2_ref,
          o_ref, g_scr, w2_scr, *, tn, num_experts, f):
    ei = pl.program_id(1)
    within = ei % (EPR + 1)
    round_idx = ei // (EPR + 1)
    col = jnp.minimum(within, EPR - 1)
    exp = round_idx * EPR + col

    @pl.when(within < EPR)
    def _first_matmul():
        onehot = (jax.lax.broadcasted_iota(jnp.int32, (1, num_experts), 1)
                  == exp).astype(jnp.float32)
        disp = jnp.sum(dp_ref[:] * onehot, axis=1, keepdims=True)
        comb = jnp.sum(cb_ref[:] * onehot, axis=1, keepdims=True)
        dmask = jnp.sum(dm_ref[:] * onehot, axis=1, keepdims=True)

        h0 = jnp.dot(x_ref[...], w1_ref[0].astype(jnp.bfloat16),
                     preferred_element_type=jnp.float32)

        mask = (dmask != 0.0).astype(jnp.float32)

        h = h0 * disp + mask * b1_ref[0, 0][None, :]
        g = (0.5 * comb) * h * (1.0 + jax.lax.erf(h * 0.7071067811865476))
        g_scr[:, pl.ds(col * f, f)] = g.astype(jnp.bfloat16)
        w2_scr[pl.ds(col * f, f), :] = w2_ref[0].astype(jnp.bfloat16)

    @pl.when(within == EPR)
    def _second_matmul():
        acc = jnp.dot(g_scr[...], w2_scr[...],
                      preferred_element_type=jnp.float32)

        @pl.when(round_idx == 0)
        def _init():
            o_ref[...] = acc + b2_ref[0][None, :]

        @pl.when(round_idx > 0)
        def _accum():
            o_ref[...] += acc


@jax.jit
def kernel(x, dispatch_tensor, combine_tensor, w1, b1, w2, b2):
    b, n, h = x.shape
    e, _, f = w1.shape
    tn = TN
    num_t = n // tn
    num_rounds = e // EPR
    steps = num_rounds * (EPR + 1)

    x2 = x.reshape(n, h)
    xb = x2.astype(jnp.bfloat16)
    dp = dispatch_tensor.reshape(n, e)
    cb = combine_tensor.reshape(n, e)
    dm = dp * jnp.sum(x2, axis=-1, keepdims=True)
    b1r = b1.reshape(e, 1, f)
    b2r = b2.reshape(1, h)

    def wmap(ti, ei):
        exp = (ei // (EPR + 1)) * EPR + jnp.minimum(ei % (EPR + 1), EPR - 1)
        return (exp, 0, 0)

    out = pl.pallas_call(
        functools.partial(_body, tn=tn, num_experts=e, f=f),
        grid=(num_t, steps),
        in_specs=[
            pl.BlockSpec((tn, h), lambda ti, ei: (ti, 0)),
            pl.BlockSpec((tn, e), lambda ti, ei: (ti, 0)),
            pl.BlockSpec((tn, e), lambda ti, ei: (ti, 0)),
            pl.BlockSpec((tn, e), lambda ti, ei: (ti, 0)),
            pl.BlockSpec((1, h, f), wmap),
            pl.BlockSpec((1, 1, f), wmap),
            pl.BlockSpec((1, f, h), wmap),
            pl.BlockSpec((1, h), lambda ti, ei: (0, 0)),
        ],
        out_specs=pl.BlockSpec((tn, h), lambda ti, ei: (ti, 0)),
        out_shape=jax.ShapeDtypeStruct((n, h), jnp.float32),
        scratch_shapes=[
            pltpu.VMEM((tn, EPR * f), jnp.bfloat16),
            pltpu.VMEM((EPR * f, h), jnp.bfloat16),
        ],
        compiler_params=pltpu.CompilerParams(
            dimension_semantics=("arbitrary", "arbitrary"),
        ),
    )(xb, dp, cb, dm, w1, b1r, w2, b2r)

    return out.reshape(b, n, h)
